# Initial kernel scaffold; baseline (speedup 1.0000x reference)
import jax
import jax.numpy as jnp
from jax.experimental import pallas as pl


def kernel(sent, mask, W_word, target_emb, W_mask):
    raise NotImplementedError("write your pallas kernel here")


# speedup vs baseline: 2.5672x; 2.5672x over previous
import jax
import jax.numpy as jnp
from jax import lax
from jax.experimental import pallas as pl
from jax.experimental.pallas import tpu as pltpu
from jax.experimental.pallas import tpu_sc as plsc

_B = 4096
_L = 50
_D = 128
_MD = 50
_OUT = _D + _MD
_N = _B * _L
_NW = 32
_BPW = _B // _NW
_CB = 2
_CR = _CB * _L
_NCH = _BPW // _CB
_PW = _BPW * _L
_BIG = 1 << 20


def _body(sent_r, mask_r, w_word, tgt_e, wm_pad, out, idx_v, mask_v,
          gath_v, out_v, tv_v, wm_v, sem):
  wid = lax.axis_index("s") * 2 + lax.axis_index("c")
  pltpu.sync_copy(sent_r.at[wid], idx_v)
  pltpu.sync_copy(mask_r.at[wid], mask_v.at[pl.ds(0, _PW)])
  pltpu.sync_copy(tgt_e, tv_v)
  pltpu.sync_copy(wm_pad, wm_v)
  base = wid * _PW

  def chunk(c, _):
    coff = c * _CR
    pltpu.async_copy(w_word.at[idx_v.at[c]], gath_v, sem).wait()

    def row(i, _):
      ro = i * _OUT
      for j in range(_D // 16):
        out_v[pl.ds(ro + 16 * j, 16)] = gath_v[i, 16 * j:16 * j + 16]
      m = mask_v[pl.ds(coff + i, 16)][0]
      sel = m > 0
      for off in (0, 16, 32, 34):
        w0 = wm_v[off:off + 16]
        w1 = wm_v[64 + off:64 + off + 16]
        out_v[pl.ds(ro + _D + off, 16)] = jnp.where(sel, w1, w0)
      return 0

    lax.fori_loop(0, _CR, row, 0)

    for bl in range(_CB):
      o = coff + bl * _L
      tpos = jnp.int32(0)
      for j in reversed(range(4)):
        mv = mask_v[pl.ds(o + 16 * j, 16)]
        pos = lax.iota(jnp.int32, 16) + 16 * j
        ok = (mv > 0) & (pos < _L)
        f = plsc.all_reduce_ffs(ok)[0]
        tpos = jnp.where(f < 16, 16 * j + f, tpos)
      ro = (bl * _L + tpos) * _OUT
      for j in range(_D // 16):
        out_v[pl.ds(ro + 16 * j, 16)] = tv_v[16 * j:16 * j + 16]

    pltpu.sync_copy(out_v, out.at[pl.ds((base + coff) * _OUT, _CR * _OUT)])
    return 0

  lax.fori_loop(0, _NCH, chunk, 0)


@jax.jit
def _run(sent_r, mask_r, w_word, tgt_e, wm_pad):
  mesh = plsc.VectorSubcoreMesh(core_axis_name="c", subcore_axis_name="s")
  f = pl.kernel(
      _body,
      out_type=jax.ShapeDtypeStruct((_N * _OUT,), jnp.float32),
      mesh=mesh,
      compiler_params=pltpu.CompilerParams(needs_layout_passes=False),
      scratch_types=[
          pltpu.VMEM((_NCH, _CR), jnp.int32),
          pltpu.VMEM((_PW + 16,), jnp.int32),
          pltpu.VMEM((_CR, _D), jnp.float32),
          pltpu.VMEM((_CR * _OUT,), jnp.float32),
          pltpu.VMEM((_D,), jnp.float32),
          pltpu.VMEM((128,), jnp.float32),
          pltpu.SemaphoreType.DMA,
      ],
  )
  return f(sent_r, mask_r, w_word, tgt_e, wm_pad)


def kernel(sent, mask, W_word, target_emb, W_mask):
  sent_r = sent.reshape(_NW, _NCH, _CR)
  mask_r = mask.reshape(_NW, _PW)
  wm_pad = jnp.pad(W_mask, ((0, 0), (0, 64 - _MD))).reshape(-1)
  out = _run(sent_r, mask_r, W_word, target_emb, wm_pad)
  return out.reshape(_B, _L, _OUT)


# speedup vs baseline: 4.2615x; 1.6600x over previous
import jax
import jax.numpy as jnp
from jax import lax
from jax.experimental import pallas as pl
from jax.experimental.pallas import tpu as pltpu
from jax.experimental.pallas import tpu_sc as plsc

_B = 4096
_L = 50
_D = 128
_MD = 50
_OUT = _D + _MD
_NW = 32
_BPW = _B // _NW
_CB = 2
_CR = _CB * _L
_NCH = _BPW // _CB
_PW = _BPW * _L


def _assemble(out_v, gath_v, mask_v, tv_v, wm_v, coff):
  for bl in range(_CB):

    def row(rr, _):
      i = bl * _L + rr
      for j in range(_D // 16):
        out_v[bl, rr, 16 * j:16 * j + 16] = gath_v[i, 16 * j:16 * j + 16]
      m = mask_v[pl.ds(coff + i, 16)][0]
      sel = m > 0
      for off in (0, 16, 32, 34):
        w0 = wm_v[off:off + 16]
        w1 = wm_v[64 + off:64 + off + 16]
        out_v[bl, rr, _D + off:_D + off + 16] = jnp.where(sel, w1, w0)
      return 0

    lax.fori_loop(0, _L, row, 0)

    o = coff + bl * _L
    tpos = jnp.int32(0)
    for j in reversed(range(4)):
      mv = mask_v[pl.ds(o + 16 * j, 16)]
      pos = lax.iota(jnp.int32, 16) + 16 * j
      ok = (mv > 0) & (pos < _L)
      f = plsc.all_reduce_ffs(ok)[0]
      tpos = jnp.where(f < 16, 16 * j + f, tpos)
    for j in range(_D // 16):
      out_v[bl, tpos, 16 * j:16 * j + 16] = tv_v[16 * j:16 * j + 16]


def _body(sent_r, mask_r, w_word, tgt_e, wm_pad, out, idx_v, mask_v, tv_v,
          wm_v, gath_a, gath_b, out_a, out_b, sem_ga, sem_gb, sem_oa, sem_ob):
  wid = lax.axis_index("s") * 2 + lax.axis_index("c")
  pltpu.sync_copy(sent_r.at[wid], idx_v)
  pltpu.sync_copy(mask_r.at[wid], mask_v.at[pl.ds(0, _PW)])
  pltpu.sync_copy(tgt_e, tv_v)
  pltpu.sync_copy(wm_pad, wm_v)
  gath = (gath_a, gath_b)
  sem_g = (sem_ga, sem_gb)
  out_v = (out_a, out_b)
  sem_o = (sem_oa, sem_ob)
  bbase = wid * _BPW

  def gather(c, par):
    pltpu.async_copy(w_word.at[idx_v.at[c]], gath[par], sem_g[par])

  def wait_gather(par):
    pltpu.make_async_copy(w_word.at[idx_v.at[0]], gath[par], sem_g[par]).wait()

  def put(c, par):
    pltpu.async_copy(out_v[par], out.at[pl.ds(bbase + c * _CB, _CB)],
                     sem_o[par])

  def wait_put(par):
    pltpu.make_async_copy(out_v[par], out.at[pl.ds(bbase, _CB)],
                          sem_o[par]).wait()

  gather(0, 0)
  gather(1, 1)
  for c in range(2):
    wait_gather(c)
    _assemble(out_v[c], gath[c], mask_v, tv_v, wm_v, c * _CR)
    put(c, c)
    gather(c + 2, c)

  def step(c2, _):
    for par in range(2):
      c = 2 * c2 + par
      wait_gather(par)
      wait_put(par)
      _assemble(out_v[par], gath[par], mask_v, tv_v, wm_v, c * _CR)
      put(c, par)

      @pl.when(c + 2 < _NCH)
      def _():
        gather(c + 2, par)

    return 0

  lax.fori_loop(1, _NCH // 2, step, 0)
  wait_put(0)
  wait_put(1)


@jax.jit
def _run(sent_r, mask_r, w_word, tgt_e, wm_pad):
  mesh = plsc.VectorSubcoreMesh(core_axis_name="c", subcore_axis_name="s")
  f = pl.kernel(
      _body,
      out_type=jax.ShapeDtypeStruct((_B, _L, _OUT), jnp.float32),
      mesh=mesh,
      compiler_params=pltpu.CompilerParams(needs_layout_passes=False),
      scratch_types=[
          pltpu.VMEM((_NCH, _CR), jnp.int32),
          pltpu.VMEM((_PW + 16,), jnp.int32),
          pltpu.VMEM((_D,), jnp.float32),
          pltpu.VMEM((128,), jnp.float32),
          pltpu.VMEM((_CR, _D), jnp.float32),
          pltpu.VMEM((_CR, _D), jnp.float32),
          pltpu.VMEM((_CB, _L, _OUT), jnp.float32),
          pltpu.VMEM((_CB, _L, _OUT), jnp.float32),
          pltpu.SemaphoreType.DMA,
          pltpu.SemaphoreType.DMA,
          pltpu.SemaphoreType.DMA,
          pltpu.SemaphoreType.DMA,
      ],
  )
  return f(sent_r, mask_r, w_word, tgt_e, wm_pad)


def kernel(sent, mask, W_word, target_emb, W_mask):
  sent_r = sent.reshape(_NW, _NCH, _CR)
  mask_r = mask.reshape(_NW, _PW)
  wm_pad = jnp.pad(W_mask, ((0, 0), (0, 64 - _MD))).reshape(-1)
  return _run(sent_r, mask_r, W_word, target_emb, wm_pad)


# speedup vs baseline: 5.0050x; 1.1745x over previous
import jax
import jax.numpy as jnp
from jax import lax
from jax.experimental import pallas as pl
from jax.experimental.pallas import tpu as pltpu
from jax.experimental.pallas import tpu_sc as plsc

_B = 4096
_L = 50
_D = 128
_MD = 50
_OUT = _D + _MD
_NW = 32
_CB = 2
_CR = _CB * _L
_SPLIT = 1


def _assemble(out_v, mask_v, tv_v, wm_v, coff):
  for bl in range(_CB):

    def row(rr, _):
      m = mask_v[pl.ds(coff + bl * _L + rr, 16)][0]
      sel = m > 0
      for off in (0, 16, 32, 34):
        w0 = wm_v[off:off + 16]
        w1 = wm_v[64 + off:64 + off + 16]
        out_v[bl, rr, _D + off:_D + off + 16] = jnp.where(sel, w1, w0)
      return 0

    lax.fori_loop(0, _L, row, 0)

    o = coff + bl * _L
    tpos = jnp.int32(0)
    for j in reversed(range(4)):
      mv = mask_v[pl.ds(o + 16 * j, 16)]
      pos = lax.iota(jnp.int32, 16) + 16 * j
      ok = (mv > 0) & (pos < _L)
      f = plsc.all_reduce_ffs(ok)[0]
      tpos = jnp.where(f < 16, 16 * j + f, tpos)
    for j in range(_D // 16):
      out_v[bl, tpos, 16 * j:16 * j + 16] = tv_v[16 * j:16 * j + 16]


def _body(sent_r, mask_r, w_word, tgt_e, wm_pad, out, idx_v, mask_v, tv_v,
          wm_v, out_a, out_b, sem_ga, sem_gb, sem_oa, sem_ob):
  nch = sent_r.shape[1]
  pw = nch * _CR
  wid = lax.axis_index("s") * 2 + lax.axis_index("c")
  pltpu.sync_copy(sent_r.at[wid], idx_v)
  pltpu.sync_copy(mask_r.at[wid], mask_v.at[pl.ds(0, pw)])
  pltpu.sync_copy(tgt_e, tv_v)
  pltpu.sync_copy(wm_pad, wm_v)
  out_v = (out_a, out_b)
  sem_g = (sem_ga, sem_gb)
  sem_o = (sem_oa, sem_ob)
  bbase = wid * nch * _CB

  def gather(c, par):
    for bl in range(_CB):
      pltpu.async_copy(w_word.at[idx_v.at[c, bl]],
                       out_v[par].at[bl, :, pl.ds(0, _D)], sem_g[par])

  def wait_gather(par):
    for bl in range(_CB):
      pltpu.make_async_copy(w_word.at[idx_v.at[0, bl]],
                            out_v[par].at[bl, :, pl.ds(0, _D)],
                            sem_g[par]).wait()

  def put(c, par):
    pltpu.async_copy(out_v[par], out.at[pl.ds(bbase + c * _CB, _CB)],
                     sem_o[par])

  def wait_put(par):
    pltpu.make_async_copy(out_v[par], out.at[pl.ds(bbase, _CB)],
                          sem_o[par]).wait()

  gather(0, 0)
  gather(1, 1)
  wait_gather(0)
  _assemble(out_v[0], mask_v, tv_v, wm_v, 0)
  put(0, 0)

  def step(c, _):
    par = lax.rem(c, 2)

    def iteration(par):
      wait_gather(par)
      _assemble(out_v[par], mask_v, tv_v, wm_v, c * _CR)
      put(c, par)
      wait_put(1 - par)

      @pl.when(c + 1 < nch)
      def _():
        gather(c + 1, 1 - par)

    lax.cond(par == 0, lambda: iteration(0), lambda: iteration(1))
    return 0

  lax.fori_loop(1, nch, step, 0)
  wait_put((nch - 1) % 2)


def _run(sent_r, mask_r, w_word, tgt_e, wm_pad):
  nch = sent_r.shape[1]
  nb = _NW * nch * _CB
  mesh = plsc.VectorSubcoreMesh(core_axis_name="c", subcore_axis_name="s")
  f = pl.kernel(
      _body,
      out_type=jax.ShapeDtypeStruct((nb, _L, _OUT), jnp.float32),
      mesh=mesh,
      compiler_params=pltpu.CompilerParams(needs_layout_passes=False),
      scratch_types=[
          pltpu.VMEM((nch, _CB, _L), jnp.int32),
          pltpu.VMEM((nch * _CR + 16,), jnp.int32),
          pltpu.VMEM((_D,), jnp.float32),
          pltpu.VMEM((128,), jnp.float32),
          pltpu.VMEM((_CB, _L, _OUT), jnp.float32),
          pltpu.VMEM((_CB, _L, _OUT), jnp.float32),
          pltpu.SemaphoreType.DMA,
          pltpu.SemaphoreType.DMA,
          pltpu.SemaphoreType.DMA,
          pltpu.SemaphoreType.DMA,
      ],
  )
  return f(sent_r, mask_r, w_word, tgt_e, wm_pad)


@jax.jit
def _run_all(sent, mask, W_word, target_emb, W_mask):
  bp = _B // _SPLIT
  nch = bp // (_NW * _CB)
  wm_pad = jnp.pad(W_mask, ((0, 0), (0, 64 - _MD))).reshape(-1)
  pieces = []
  for s in range(_SPLIT):
    sent_s = lax.slice_in_dim(sent, s * bp, (s + 1) * bp, axis=0)
    mask_s = lax.slice_in_dim(mask, s * bp, (s + 1) * bp, axis=0)
    sent_r = sent_s.reshape(_NW, nch, _CB, _L)
    mask_r = mask_s.reshape(_NW, nch * _CR)
    pieces.append(_run(sent_r, mask_r, W_word, target_emb, wm_pad))
  return jnp.concatenate(pieces, axis=0)


def kernel(sent, mask, W_word, target_emb, W_mask):
  return _run_all(sent, mask, W_word, target_emb, W_mask)


# speedup vs baseline: 5.8795x; 1.1747x over previous
import jax
import jax.numpy as jnp
from jax import lax
from jax.experimental import pallas as pl
from jax.experimental.pallas import tpu as pltpu
from jax.experimental.pallas import tpu_sc as plsc

_B = 4096
_L = 50
_D = 128
_MD = 50
_OUT = _D + _MD
_NW = 32
_CB = 2
_CR = _CB * _L
_TOFF = (0, 16, 32, 34)


def _assemble(out_v, mask_v, tvecs, w0s, w1s, coff):
  for bl in range(_CB):

    def row(rr, _):
      m = mask_v[pl.ds(coff + bl * _L + rr, 16)][0]
      sel = m > 0
      for k, off in enumerate(_TOFF):
        out_v[bl, rr, _D + off:_D + off + 16] = jnp.where(
            sel, w1s[k], w0s[k])
      return 0

    lax.fori_loop(0, _L, row, 0)

    o = coff + bl * _L
    tpos = jnp.int32(0)
    for j in reversed(range(4)):
      mv = mask_v[pl.ds(o + 16 * j, 16)]
      pos = lax.iota(jnp.int32, 16) + 16 * j
      ok = (mv > 0) & (pos < _L)
      f = plsc.all_reduce_ffs(ok)[0]
      tpos = jnp.where(f < 16, 16 * j + f, tpos)
    for j in range(_D // 16):
      out_v[bl, tpos, 16 * j:16 * j + 16] = tvecs[j]


def _body(sent_r, mask_r, w_word, tgt_e, wm_pad, out, idx_v, mask_v, tv_v,
          wm_v, out_a, out_b, sem_ga, sem_gb, sem_oa, sem_ob):
  nch = sent_r.shape[1]
  pw = nch * _CR
  wid = lax.axis_index("s") * 2 + lax.axis_index("c")
  pltpu.sync_copy(sent_r.at[wid], idx_v)
  pltpu.sync_copy(mask_r.at[wid], mask_v.at[pl.ds(0, pw)])
  pltpu.sync_copy(tgt_e, tv_v)
  pltpu.sync_copy(wm_pad, wm_v)
  out_v = (out_a, out_b)
  sem_g = (sem_ga, sem_gb)
  sem_o = (sem_oa, sem_ob)
  bbase = wid * nch * _CB

  tvecs = [tv_v[16 * j:16 * j + 16] for j in range(_D // 16)]
  w0s = [wm_v[off:off + 16] for off in _TOFF]
  w1s = [wm_v[64 + off:64 + off + 16] for off in _TOFF]

  def gather(c, par):
    for bl in range(_CB):
      pltpu.async_copy(w_word.at[idx_v.at[c, bl]],
                       out_v[par].at[bl, :, pl.ds(0, _D)], sem_g[par])

  def wait_gather(par):
    for bl in range(_CB):
      pltpu.make_async_copy(w_word.at[idx_v.at[0, bl]],
                            out_v[par].at[bl, :, pl.ds(0, _D)],
                            sem_g[par]).wait()

  def put(c, par):
    pltpu.async_copy(out_v[par], out.at[pl.ds(bbase + c * _CB, _CB)],
                     sem_o[par])

  def wait_put(par):
    pltpu.make_async_copy(out_v[par], out.at[pl.ds(bbase, _CB)],
                          sem_o[par]).wait()

  gather(0, 0)
  gather(1, 1)
  wait_gather(0)
  _assemble(out_v[0], mask_v, tvecs, w0s, w1s, 0)
  put(0, 0)

  def step(c, _):

    def iteration(par):
      wait_gather(par)
      _assemble(out_v[par], mask_v, tvecs, w0s, w1s, c * _CR)
      put(c, par)
      wait_put(1 - par)

      @pl.when(c + 1 < nch)
      def _():
        gather(c + 1, 1 - par)

    lax.cond(lax.rem(c, 2) == 0, lambda: iteration(0), lambda: iteration(1))
    return 0

  lax.fori_loop(1, nch, step, 0)
  wait_put((nch - 1) % 2)


def _run(sent_r, mask_r, w_word, tgt_e, wm_pad):
  nch = sent_r.shape[1]
  nb = _NW * nch * _CB
  mesh = plsc.VectorSubcoreMesh(core_axis_name="c", subcore_axis_name="s")
  f = pl.kernel(
      _body,
      out_type=jax.ShapeDtypeStruct((nb, _L, _OUT), jnp.float32),
      mesh=mesh,
      compiler_params=pltpu.CompilerParams(needs_layout_passes=False),
      scratch_types=[
          pltpu.VMEM((nch, _CB, _L), jnp.int32),
          pltpu.VMEM((nch * _CR + 16,), jnp.int32),
          pltpu.VMEM((_D,), jnp.float32),
          pltpu.VMEM((128,), jnp.float32),
          pltpu.VMEM((_CB, _L, _OUT), jnp.float32),
          pltpu.VMEM((_CB, _L, _OUT), jnp.float32),
          pltpu.SemaphoreType.DMA,
          pltpu.SemaphoreType.DMA,
          pltpu.SemaphoreType.DMA,
          pltpu.SemaphoreType.DMA,
      ],
  )
  return f(sent_r, mask_r, w_word, tgt_e, wm_pad)


@jax.jit
def _run_all(sent, mask, W_word, target_emb, W_mask):
  nch = _B // (_NW * _CB)
  sent_r = sent.reshape(_NW, nch, _CB, _L)
  mask_r = mask.reshape(_NW, nch * _CR)
  wm_pad = jnp.pad(W_mask, ((0, 0), (0, 64 - _MD))).reshape(-1)
  return _run(sent_r, mask_r, W_word, target_emb, wm_pad)


def kernel(sent, mask, W_word, target_emb, W_mask):
  return _run_all(sent, mask, W_word, target_emb, W_mask)


# speedup vs baseline: 5.8831x; 1.0006x over previous
import jax
import jax.numpy as jnp
from jax import lax
from jax.experimental import pallas as pl
from jax.experimental.pallas import tpu as pltpu
from jax.experimental.pallas import tpu_sc as plsc

_B = 4096
_L = 50
_D = 128
_MD = 50
_OUT = _D + _MD
_NW = 32
_BPW = _B // _NW
_CR = 100
_TOFF = (0, 16, 32, 34)
_SPLIT = 1


def _make_body(ls, lsub, cb, nch):
  def _assemble(out_v, mask_v, tvecs, w0s, w1s, c):
    for bi in range(cb):
      bg = c * cb + bi

      def row(rr, _):
        m = mask_v[pl.ds(bg * _L + ls + rr, 16)][0]
        sel = m > 0
        for k, off in enumerate(_TOFF):
          out_v[bi, rr, _D + off:_D + off + 16] = jnp.where(
              sel, w1s[k], w0s[k])
        return 0

      lax.fori_loop(0, lsub, row, 0)

      o = bg * _L
      tpos = jnp.int32(0)
      for j in reversed(range(4)):
        mv = mask_v[pl.ds(o + 16 * j, 16)]
        pos = lax.iota(jnp.int32, 16) + 16 * j
        ok = (mv > 0) & (pos < _L)
        f = plsc.all_reduce_ffs(ok)[0]
        tpos = jnp.where(f < 16, 16 * j + f, tpos)

      @pl.when((tpos >= ls) & (tpos < ls + lsub))
      def _():
        for j in range(_D // 16):
          out_v[bi, tpos - ls, 16 * j:16 * j + 16] = tvecs[j]

  def _body(sent_r, mask_r, w_word, tgt_e, wm_pad, out, idx_v, mask_v, tv_v,
            wm_v, out_a, out_b, sem_ga, sem_gb, sem_oa, sem_ob):
    wid = lax.axis_index("s") * 2 + lax.axis_index("c")
    pltpu.sync_copy(sent_r.at[wid], idx_v)
    pltpu.sync_copy(mask_r.at[wid], mask_v.at[pl.ds(0, _BPW * _L)])
    pltpu.sync_copy(tgt_e, tv_v)
    pltpu.sync_copy(wm_pad, wm_v)
    out_v = (out_a, out_b)
    sem_g = (sem_ga, sem_gb)
    sem_o = (sem_oa, sem_ob)
    bbase = wid * _BPW

    tvecs = [tv_v[16 * j:16 * j + 16] for j in range(_D // 16)]
    w0s = [wm_v[off:off + 16] for off in _TOFF]
    w1s = [wm_v[64 + off:64 + off + 16] for off in _TOFF]

    def gather(c, par):
      for bi in range(cb):
        pltpu.async_copy(w_word.at[idx_v.at[c, bi]],
                         out_v[par].at[bi, :, pl.ds(0, _D)], sem_g[par])

    def wait_gather(par):
      for bi in range(cb):
        pltpu.make_async_copy(w_word.at[idx_v.at[0, bi]],
                              out_v[par].at[bi, :, pl.ds(0, _D)],
                              sem_g[par]).wait()

    def put(c, par):
      pltpu.async_copy(out_v[par], out.at[pl.ds(bbase + c * cb, cb)],
                       sem_o[par])

    def wait_put(par):
      pltpu.make_async_copy(out_v[par], out.at[pl.ds(bbase, cb)],
                            sem_o[par]).wait()

    gather(0, 0)
    gather(1, 1)
    wait_gather(0)
    _assemble(out_v[0], mask_v, tvecs, w0s, w1s, 0)
    put(0, 0)

    def step(c, _):

      def iteration(par):
        wait_gather(par)
        _assemble(out_v[par], mask_v, tvecs, w0s, w1s, c)
        put(c, par)
        wait_put(1 - par)

        @pl.when(c + 1 < nch)
        def _():
          gather(c + 1, 1 - par)

      lax.cond(lax.rem(c, 2) == 0, lambda: iteration(0), lambda: iteration(1))
      return 0

    lax.fori_loop(1, nch, step, 0)
    wait_put((nch - 1) % 2)

  return _body


def _run(sent_r, mask_r, w_word, tgt_e, wm_pad, ls):
  nch, cb, lsub = sent_r.shape[1], sent_r.shape[2], sent_r.shape[3]
  mesh = plsc.VectorSubcoreMesh(core_axis_name="c", subcore_axis_name="s")
  f = pl.kernel(
      _make_body(ls, lsub, cb, nch),
      out_type=jax.ShapeDtypeStruct((_B, lsub, _OUT), jnp.float32),
      mesh=mesh,
      compiler_params=pltpu.CompilerParams(needs_layout_passes=False),
      scratch_types=[
          pltpu.VMEM((nch, cb, lsub), jnp.int32),
          pltpu.VMEM((_BPW * _L + 16,), jnp.int32),
          pltpu.VMEM((_D,), jnp.float32),
          pltpu.VMEM((128,), jnp.float32),
          pltpu.VMEM((cb, lsub, _OUT), jnp.float32),
          pltpu.VMEM((cb, lsub, _OUT), jnp.float32),
          pltpu.SemaphoreType.DMA,
          pltpu.SemaphoreType.DMA,
          pltpu.SemaphoreType.DMA,
          pltpu.SemaphoreType.DMA,
      ],
  )
  return f(sent_r, mask_r, w_word, tgt_e, wm_pad)


@jax.jit
def _run_all(sent, mask, W_word, target_emb, W_mask):
  lsub = _L // _SPLIT
  cb = _CR // lsub
  nch = _BPW // cb
  wm_pad = jnp.pad(W_mask, ((0, 0), (0, 64 - _MD))).reshape(-1)
  mask_r = mask.reshape(_NW, _BPW * _L)
  pieces = []
  for s in range(_SPLIT):
    ls = s * lsub
    sent_s = lax.slice_in_dim(sent, ls, ls + lsub, axis=1)
    sent_r = sent_s.reshape(_NW, nch, cb, lsub)
    pieces.append(_run(sent_r, mask_r, W_word, target_emb, wm_pad, ls))
  if _SPLIT == 1:
    return pieces[0]
  return jnp.concatenate(pieces, axis=1)


def kernel(sent, mask, W_word, target_emb, W_mask):
  return _run_all(sent, mask, W_word, target_emb, W_mask)
